# baseline (speedup 1.0000x reference)
import functools

import jax
import jax.numpy as jnp
from jax.experimental import pallas as pl
from jax.experimental.pallas import tpu as pltpu


def _scse_kernel(x_ref, w1d_ref, b1_ref, w2d_ref, b2_ref, wcd_ref, bc_ref,
                 o_ref, *, inv_hw):
    bt, c, hwp = x_ref.shape
    x3 = x_ref[...]
    x = x3.reshape(bt * c, hwp)

    s = jnp.sum(x, axis=1, keepdims=True) * inv_hw
    h = jnp.dot(w1d_ref[...], s, preferred_element_type=jnp.float32)
    h = jnp.maximum(h + b1_ref[...], 0.0)
    z = jnp.dot(w2d_ref[...], h, preferred_element_type=jnp.float32)
    gate_c = jax.nn.sigmoid(z + b2_ref[...])

    sz = jnp.dot(wcd_ref[...], x, preferred_element_type=jnp.float32)
    gate_s = jax.nn.sigmoid(sz + bc_ref[0])

    o_ref[...] = x3 * (gate_c.reshape(bt, c, 1) + gate_s.reshape(bt, 1, hwp))


def _block_diag(w, bt):
    m, n = w.shape
    out = jnp.zeros((bt, m, bt, n), w.dtype)
    idx = jnp.arange(bt)
    out = out.at[idx, :, idx, :].set(jnp.broadcast_to(w, (bt, m, n)))
    return out.reshape(bt * m, bt * n)


def kernel(x, w1, b1, w2, b2, wconv, bconv):
    B, C, H, W = x.shape
    HW = H * W
    HWP = 896
    Cr = w1.shape[0]
    Bt = 4

    x_flat = x.reshape(B, C, HW)
    x_pad = jnp.pad(x_flat, ((0, 0), (0, 0), (0, HWP - HW)))
    w1d = _block_diag(w1, Bt)
    w2d = _block_diag(w2, Bt)
    wcd = _block_diag(wconv.reshape(1, C), Bt)
    b1t = jnp.tile(b1, Bt).reshape(Bt * Cr, 1)
    b2t = jnp.tile(b2, Bt).reshape(Bt * C, 1)
    bc_vec = bconv.reshape(1)

    kernel_fn = functools.partial(_scse_kernel, inv_hw=1.0 / HW)
    out_pad = pl.pallas_call(
        kernel_fn,
        out_shape=jax.ShapeDtypeStruct((B, C, HWP), x.dtype),
        grid=(B // Bt,),
        in_specs=[
            pl.BlockSpec((Bt, C, HWP), lambda i: (i, 0, 0)),
            pl.BlockSpec((Bt * Cr, Bt * C), lambda i: (0, 0)),
            pl.BlockSpec((Bt * Cr, 1), lambda i: (0, 0)),
            pl.BlockSpec((Bt * C, Bt * Cr), lambda i: (0, 0)),
            pl.BlockSpec((Bt * C, 1), lambda i: (0, 0)),
            pl.BlockSpec((Bt, Bt * C), lambda i: (0, 0)),
            pl.BlockSpec(memory_space=pltpu.MemorySpace.SMEM),
        ],
        out_specs=pl.BlockSpec((Bt, C, HWP), lambda i: (i, 0, 0)),
        compiler_params=pltpu.CompilerParams(
            dimension_semantics=("parallel",),
            vmem_limit_bytes=56 * 1024 * 1024),
    )(x_pad, w1d, b1t, w2d, b2t, wcd, bc_vec)

    return out_pad[:, :, :HW].reshape(B, C, H, W)


# speedup vs baseline: 1.1498x; 1.1498x over previous
import functools

import jax
import jax.numpy as jnp
from jax.experimental import pallas as pl
from jax.experimental.pallas import tpu as pltpu

NBUF = 4
BT = 4
LANES = 784


def _scse_kernel(x_hbm, w1d_ref, b1_ref, w2d_ref, b2_ref, wcd_ref, bc_ref,
                 o_hbm, xbuf, obuf, in_sem, out_sem, *, inv_hw, n_rows, c):
    rb = BT * c
    x2 = x_hbm.reshape(n_rows, LANES)
    o2 = o_hbm.reshape(n_rows, LANES)
    n_steps = n_rows // rb

    def dma_in(slot, step):
        return pltpu.make_async_copy(
            x2.at[pl.ds(pl.multiple_of(step * rb, rb), rb)],
            xbuf.at[slot], in_sem.at[slot])

    def dma_out(slot, step):
        return pltpu.make_async_copy(
            obuf.at[slot],
            o2.at[pl.ds(pl.multiple_of(step * rb, rb), rb)], out_sem.at[slot])

    for i in range(NBUF):
        dma_in(i, i).start()

    def body(step, _):
        slot = jax.lax.rem(step, NBUF)
        dma_in(slot, step).wait()

        @pl.when(step >= NBUF)
        def _():
            dma_out(slot, step - NBUF).wait()

        x = xbuf[slot]

        s = jnp.sum(x, axis=1, keepdims=True)
        h = jnp.dot(w1d_ref[...], s, preferred_element_type=jnp.float32)
        h = jnp.maximum(h + b1_ref[...], 0.0)
        z = jnp.dot(w2d_ref[...], h, preferred_element_type=jnp.float32)
        gate_c = jax.nn.sigmoid(z + b2_ref[...])

        sz = jnp.dot(wcd_ref[...], x, preferred_element_type=jnp.float32)
        gate_s = jax.nn.sigmoid(sz + bc_ref[0])

        x3 = x.reshape(BT, c, LANES)
        out3 = x3 * (gate_c.reshape(BT, c, 1) + gate_s.reshape(BT, 1, LANES))
        obuf[slot] = out3.reshape(rb, LANES)

        dma_out(slot, step).start()

        @pl.when(step + NBUF < n_steps)
        def _():
            dma_in(slot, step + NBUF).start()
        return ()

    jax.lax.fori_loop(0, n_steps, body, ())

    for i in range(NBUF):
        step = n_steps - NBUF + i
        dma_out(jax.lax.rem(step, NBUF), step).wait()


def _block_diag(w, bt):
    m, n = w.shape
    out = jnp.zeros((bt, m, bt, n), w.dtype)
    idx = jnp.arange(bt)
    out = out.at[idx, :, idx, :].set(jnp.broadcast_to(w, (bt, m, n)))
    return out.reshape(bt * m, bt * n)


def kernel(x, w1, b1, w2, b2, wconv, bconv):
    B, C, H, W = x.shape
    HW = H * W
    Cr = w1.shape[0]
    n_rows = B * C
    rb = BT * C

    x_flat = x.reshape(B, C, HW)
    w1d = _block_diag(w1 * (1.0 / HW), BT)
    w2d = _block_diag(w2, BT)
    wcd = _block_diag(wconv.reshape(1, C), BT)
    b1t = jnp.tile(b1, BT).reshape(BT * Cr, 1)
    b2t = jnp.tile(b2, BT).reshape(BT * C, 1)
    bc_vec = bconv.reshape(1)

    kernel_fn = functools.partial(_scse_kernel, inv_hw=1.0 / HW,
                                  n_rows=n_rows, c=C)
    out_flat = pl.pallas_call(
        kernel_fn,
        out_shape=jax.ShapeDtypeStruct((B, C, HW), x.dtype),
        in_specs=[
            pl.BlockSpec(memory_space=pltpu.MemorySpace.HBM),
            pl.BlockSpec((BT * Cr, BT * C), lambda: (0, 0)),
            pl.BlockSpec((BT * Cr, 1), lambda: (0, 0)),
            pl.BlockSpec((BT * C, BT * Cr), lambda: (0, 0)),
            pl.BlockSpec((BT * C, 1), lambda: (0, 0)),
            pl.BlockSpec((BT, BT * C), lambda: (0, 0)),
            pl.BlockSpec(memory_space=pltpu.MemorySpace.SMEM),
        ],
        out_specs=pl.BlockSpec(memory_space=pltpu.MemorySpace.HBM),
        scratch_shapes=[
            pltpu.VMEM((NBUF, rb, LANES), jnp.float32),
            pltpu.VMEM((NBUF, rb, LANES), jnp.float32),
            pltpu.SemaphoreType.DMA((NBUF,)),
            pltpu.SemaphoreType.DMA((NBUF,)),
        ],
        compiler_params=pltpu.CompilerParams(
            vmem_limit_bytes=60 * 1024 * 1024),
    )(x_flat, w1d, b1t, w2d, b2t, wcd, bc_vec)

    return out_flat.reshape(B, C, H, W)
